# baseline (speedup 1.0000x reference)
import functools

import jax
import jax.numpy as jnp
from jax import lax
from jax.experimental import pallas as pl
from jax.experimental.pallas import tpu as pltpu
from jax.experimental.pallas import tpu_sc as plsc

NSEG = 16384
N_ATOMS = 1048576
EMB = 64
NZ = 119
ZPAD = 128
NC, NS, L = 2, 16, 16
NW = NC * NS
CHUNK = N_ATOMS // NW
NVEC = CHUNK // L
BLK = 512


UNROLL = 16


def _sc_body(att_h, w_h, sae_h, num_h, mol_h, out_h,
             att_vm, w_vm, sae_vm, tabs_vm, ctab_vm, nums_vm, mols_vm,
             acc_vm, idx_vm, shared, sem_n, sem_m):
    c = lax.axis_index("c")
    s = lax.axis_index("s")
    base = (s * NC + c) * CHUNK

    cp_n = pltpu.make_async_copy(num_h.at[pl.ds(base, CHUNK)], nums_vm, sem_n)
    cp_m = pltpu.make_async_copy(mol_h.at[pl.ds(base, CHUNK)],
                                 mols_vm.at[pl.ds(0, CHUNK)], sem_m)
    cp_n.start()
    cp_m.start()

    pltpu.sync_copy(att_h, att_vm)
    pltpu.sync_copy(w_h, w_vm)
    pltpu.sync_copy(sae_h.at[pl.ds(0, ZPAD)], sae_vm)

    iota16 = lax.iota(jnp.int32, L)

    for zb in range(ZPAD // L):
        z_vec = zb * L + iota16
        zmask = z_vec < NZ

        def dbody(d, a, _z=z_vec, _m=zmask):
            dsp = jnp.full((L,), 0, jnp.int32) + d
            ws = plsc.load_gather(w_vm, [dsp])
            return a + plsc.load_gather(att_vm, [_z, dsp], mask=_m) * ws
        v = lax.fori_loop(0, EMB, dbody, jnp.zeros((L,), jnp.float32))
        tabs_vm[pl.ds(zb * L, L)] = v + sae_vm[pl.ds(zb * L, L)]

    def rbody(z, carry):
        v = plsc.load_gather(tabs_vm, [jnp.full((L,), 0, jnp.int32) + z])
        ctab_vm[pl.ds(z * L, L)] = v
        return carry
    lax.fori_loop(0, ZPAD, rbody, 0)

    zero16 = jnp.zeros((L,), jnp.float32)

    @plsc.parallel_loop(0, NSEG // L, unroll=UNROLL)
    def _(i):
        acc_vm[pl.ds(i * L, L)] = zero16

    @pl.when(s == 0)
    def _():
        pltpu.sync_copy(acc_vm.at[pl.ds(0, NSEG)], shared)

    cp_m.wait()
    mols_vm[pl.ds(CHUNK, L)] = jnp.full((L,), NSEG, jnp.int32)
    s_lo = jnp.min(mols_vm[pl.ds(0, L)])
    s_hi = jnp.max(mols_vm[pl.ds(CHUNK - L, L)])
    lo = (s_lo // BLK) * BLK
    nblk = (s_hi - lo) // BLK + 1

    cp_n.wait()

    @plsc.parallel_loop(0, NVEC, unroll=UNROLL,
                        carry=jnp.zeros((L,), jnp.float32))
    def _(i, run):
        o = i * L
        nums = nums_vm[pl.ds(o, L)]
        mols = mols_vm[pl.ds(o, L)]
        moln = mols_vm[pl.ds(o + 1, L)]
        vals = plsc.load_gather(ctab_vm, [nums * L + iota16])
        p = plsc.cumsum(vals)
        cum = p + run
        m = mols != moln
        plsc.addupdate_scatter(acc_vm, [mols], cum, mask=m)
        plsc.addupdate_scatter(acc_vm, [moln], -cum, mask=m)
        return run + jnp.broadcast_to(p[L - 1], (L,))

    plsc.subcore_barrier()

    def cbody(j, carry):
        bj = lo + j * BLK
        for m in range(BLK // L):
            idx_vm[pl.ds(m * L, L)] = bj + m * L + iota16
        pltpu.sync_copy(acc_vm.at[pl.ds(bj, BLK)], shared.at[idx_vm], add=True)
        return carry
    lax.fori_loop(0, nblk, cbody, 0)

    plsc.subcore_barrier()

    @pl.when(s == 0)
    def _():
        pltpu.sync_copy(shared, out_h.at[c])


@functools.partial(jax.jit, static_argnames=("interpret",))
def _sc_call(att, w, sae, numbers, mol_idx, interpret=False):
    mesh = plsc.VectorSubcoreMesh(core_axis_name="c", subcore_axis_name="s",
                                  num_cores=NC, num_subcores=NS)
    f = pl.kernel(
        _sc_body,
        out_type=jax.ShapeDtypeStruct((NC, NSEG), jnp.float32),
        mesh=mesh,
        scratch_types=[
            pltpu.VMEM((NZ, EMB), jnp.float32),
            pltpu.VMEM((EMB,), jnp.float32),
            pltpu.VMEM((ZPAD,), jnp.float32),
            pltpu.VMEM((ZPAD,), jnp.float32),
            pltpu.VMEM((ZPAD * L,), jnp.float32),
            pltpu.VMEM((CHUNK,), jnp.int32),
            pltpu.VMEM((CHUNK + L,), jnp.int32),
            pltpu.VMEM((NSEG + L,), jnp.float32),
            pltpu.VMEM((BLK,), jnp.int32),
            pltpu.VMEM_SHARED((NSEG,), jnp.float32),
            pltpu.SemaphoreType.DMA,
            pltpu.SemaphoreType.DMA,
        ],
        compiler_params=pltpu.CompilerParams(needs_layout_passes=False),
        interpret=interpret,
    )
    return f(att, w, sae, numbers, mol_idx)


def kernel(numbers, mol_idx, charge, atom_table, w, sae_tensor):
    del charge
    parts = _sc_call(atom_table, w, sae_tensor, numbers, mol_idx)
    return parts[0] + parts[1]


# speedup vs baseline: 1.0394x; 1.0394x over previous
import functools

import jax
import jax.numpy as jnp
from jax import lax
from jax.experimental import pallas as pl
from jax.experimental.pallas import tpu as pltpu
from jax.experimental.pallas import tpu_sc as plsc

NSEG = 16384
N_ATOMS = 1048576
EMB = 64
NZ = 119
ZPAD = 128
NC, NS, L = 2, 16, 16
NW = NC * NS
CHUNK = N_ATOMS // NW
NVEC = CHUNK // L
BLK = 512


UNROLL = 16


def _sc_body(att_h, w_h, sae_h, num_h, mol_h, out_h,
             att_vm, w_vm, sae_vm, tabs_vm, ctab_vm, nums_vm, mols_vm,
             acc_vm, idx_vm, shared, sem_n, sem_m):
    c = lax.axis_index("c")
    s = lax.axis_index("s")
    base = (s * NC + c) * CHUNK

    cp_n = pltpu.make_async_copy(num_h.at[pl.ds(base, CHUNK)], nums_vm, sem_n)
    cp_m = pltpu.make_async_copy(mol_h.at[pl.ds(base, CHUNK)],
                                 mols_vm.at[pl.ds(0, CHUNK)], sem_m)
    cp_n.start()
    cp_m.start()

    pltpu.sync_copy(att_h, att_vm)
    pltpu.sync_copy(w_h, w_vm)
    pltpu.sync_copy(sae_h.at[pl.ds(0, ZPAD)], sae_vm)

    iota16 = lax.iota(jnp.int32, L)

    zvecs = [zb * L + iota16 for zb in range(ZPAD // L)]
    zmasks = [zv < NZ for zv in zvecs]

    def dbody(d, accs):
        dsp = jnp.full((L,), 0, jnp.int32) + d
        ws = plsc.load_gather(w_vm, [dsp])
        return tuple(
            a + plsc.load_gather(att_vm, [zv, dsp], mask=zm) * ws
            for a, zv, zm in zip(accs, zvecs, zmasks))
    accs = lax.fori_loop(
        0, EMB, dbody,
        tuple(jnp.zeros((L,), jnp.float32) for _ in range(ZPAD // L)))
    for zb in range(ZPAD // L):
        tabs_vm[pl.ds(zb * L, L)] = accs[zb] + sae_vm[pl.ds(zb * L, L)]

    def rbody(z, carry):
        v = plsc.load_gather(tabs_vm, [jnp.full((L,), 0, jnp.int32) + z])
        ctab_vm[pl.ds(z * L, L)] = v
        return carry
    lax.fori_loop(0, ZPAD, rbody, 0)

    zero16 = jnp.zeros((L,), jnp.float32)

    @plsc.parallel_loop(0, NSEG // L, unroll=UNROLL)
    def _(i):
        acc_vm[pl.ds(i * L, L)] = zero16

    @pl.when(s == 0)
    def _():
        pltpu.sync_copy(acc_vm.at[pl.ds(0, NSEG)], shared)

    cp_m.wait()
    mols_vm[pl.ds(CHUNK, L)] = jnp.full((L,), NSEG, jnp.int32)
    s_lo = jnp.min(mols_vm[pl.ds(0, L)])
    s_hi = jnp.max(mols_vm[pl.ds(CHUNK - L, L)])
    lo = (s_lo // BLK) * BLK
    nblk = (s_hi - lo) // BLK + 1

    cp_n.wait()

    @plsc.parallel_loop(0, NVEC, unroll=UNROLL,
                        carry=jnp.zeros((L,), jnp.float32))
    def _(i, run):
        o = i * L
        nums = nums_vm[pl.ds(o, L)]
        mols = mols_vm[pl.ds(o, L)]
        moln = mols_vm[pl.ds(o + 1, L)]
        vals = plsc.load_gather(ctab_vm, [nums * L + iota16])
        p = plsc.cumsum(vals)
        cum = p + run
        m = mols != moln
        plsc.addupdate_scatter(acc_vm, [mols], cum, mask=m)
        plsc.addupdate_scatter(acc_vm, [moln], -cum, mask=m)
        return run + jnp.broadcast_to(p[L - 1], (L,))

    plsc.subcore_barrier()

    def cbody(j, carry):
        bj = lo + j * BLK
        for m in range(BLK // L):
            idx_vm[pl.ds(m * L, L)] = bj + m * L + iota16
        pltpu.sync_copy(acc_vm.at[pl.ds(bj, BLK)], shared.at[idx_vm], add=True)
        return carry
    lax.fori_loop(0, nblk, cbody, 0)

    plsc.subcore_barrier()

    @pl.when(s == 0)
    def _():
        pltpu.sync_copy(shared, out_h.at[c])


@functools.partial(jax.jit, static_argnames=("interpret",))
def _sc_call(att, w, sae, numbers, mol_idx, interpret=False):
    mesh = plsc.VectorSubcoreMesh(core_axis_name="c", subcore_axis_name="s",
                                  num_cores=NC, num_subcores=NS)
    f = pl.kernel(
        _sc_body,
        out_type=jax.ShapeDtypeStruct((NC, NSEG), jnp.float32),
        mesh=mesh,
        scratch_types=[
            pltpu.VMEM((NZ, EMB), jnp.float32),
            pltpu.VMEM((EMB,), jnp.float32),
            pltpu.VMEM((ZPAD,), jnp.float32),
            pltpu.VMEM((ZPAD,), jnp.float32),
            pltpu.VMEM((ZPAD * L,), jnp.float32),
            pltpu.VMEM((CHUNK,), jnp.int32),
            pltpu.VMEM((CHUNK + L,), jnp.int32),
            pltpu.VMEM((NSEG + L,), jnp.float32),
            pltpu.VMEM((BLK,), jnp.int32),
            pltpu.VMEM_SHARED((NSEG,), jnp.float32),
            pltpu.SemaphoreType.DMA,
            pltpu.SemaphoreType.DMA,
        ],
        compiler_params=pltpu.CompilerParams(needs_layout_passes=False),
        interpret=interpret,
    )
    return f(att, w, sae, numbers, mol_idx)


def kernel(numbers, mol_idx, charge, atom_table, w, sae_tensor):
    del charge
    parts = _sc_call(atom_table, w, sae_tensor, numbers, mol_idx)
    return parts[0] + parts[1]


# speedup vs baseline: 1.0949x; 1.0534x over previous
import functools

import jax
import jax.numpy as jnp
from jax import lax
from jax.experimental import pallas as pl
from jax.experimental.pallas import tpu as pltpu
from jax.experimental.pallas import tpu_sc as plsc

NSEG = 16384
N_ATOMS = 1048576
EMB = 64
NZ = 119
ZPAD = 128
NC, NS, L = 2, 16, 16
NW = NC * NS
CHUNK = N_ATOMS // NW
NVEC = CHUNK // L
BLK = 512
PIECES = 4
PCHUNK = CHUNK // PIECES
PVEC = PCHUNK // L


UNROLL = 8


def _sc_body(att_h, w_h, sae_h, num_h, mol_h, out_h,
             att_vm, w_vm, sae_vm, ctab_vm, nums_vm, mols_vm,
             acc_vm, idx_vm, shared,
             sn0, sn1, sn2, sn3, sm0, sm1, sm2, sm3):
    c = lax.axis_index("c")
    s = lax.axis_index("s")
    base = (s * NC + c) * CHUNK

    sems_n = (sn0, sn1, sn2, sn3)
    sems_m = (sm0, sm1, sm2, sm3)
    cps_n = []
    cps_m = []
    for p in range(PIECES):
        cps_n.append(pltpu.make_async_copy(
            num_h.at[pl.ds(base + p * PCHUNK, PCHUNK)],
            nums_vm.at[pl.ds(p * PCHUNK, PCHUNK)], sems_n[p]))
        cps_m.append(pltpu.make_async_copy(
            mol_h.at[pl.ds(base + p * PCHUNK, PCHUNK)],
            mols_vm.at[pl.ds(p * PCHUNK, PCHUNK)], sems_m[p]))
        cps_n[p].start()
        cps_m[p].start()

    pltpu.sync_copy(att_h, att_vm)
    pltpu.sync_copy(w_h, w_vm)
    pltpu.sync_copy(sae_h, sae_vm)

    accs = [jnp.zeros((L,), jnp.float32) for _ in range(ZPAD // L)]
    for db in range(EMB // L):
        wv = w_vm[pl.ds(db * L, L)]
        for j in range(L):
            ws = wv[j]
            d = db * L + j
            for zb in range(ZPAD // L):
                accs[zb] = accs[zb] + att_vm[d, pl.ds(zb * L, L)] * ws
    for zb in range(ZPAD // L):
        v = accs[zb] + sae_vm[pl.ds(zb * L, L)]
        for j in range(L):
            ctab_vm[pl.ds((zb * L + j) * L, L)] = jnp.broadcast_to(v[j], (L,))

    zero16 = jnp.zeros((L,), jnp.float32)

    @plsc.parallel_loop(0, NSEG // L, unroll=UNROLL)
    def _(i):
        acc_vm[pl.ds(i * L, L)] = zero16

    @pl.when(s == 0)
    def _():
        pltpu.sync_copy(acc_vm.at[pl.ds(0, NSEG)], shared)

    iota16 = lax.iota(jnp.int32, L)

    def body(i, run):
        o = i * L
        nums = nums_vm[pl.ds(o, L)]
        mols = mols_vm[pl.ds(o, L)]
        moln = mols_vm[pl.ds(o + 1, L)]
        vals = plsc.load_gather(ctab_vm, [nums * L + iota16])
        p = plsc.cumsum(vals)
        cum = p + run
        m = mols != moln
        plsc.addupdate_scatter(acc_vm, [mols], cum, mask=m)
        plsc.addupdate_scatter(acc_vm, [moln], -cum, mask=m)
        return run + jnp.broadcast_to(p[L - 1], (L,))

    cps_m[0].wait()
    s_lo = jnp.min(mols_vm[pl.ds(0, L)])
    lo = (s_lo // BLK) * BLK

    run = jnp.zeros((L,), jnp.float32)
    for p in range(PIECES):
        if p + 1 < PIECES:
            cps_m[p + 1].wait()
        else:
            mols_vm[pl.ds(CHUNK, L)] = jnp.full((L,), NSEG, jnp.int32)
        cps_n[p].wait()
        run = plsc.parallel_loop(p * PVEC, (p + 1) * PVEC, unroll=UNROLL,
                                 carry=run)(body)

    s_hi = jnp.max(mols_vm[pl.ds(CHUNK - L, L)])
    nblk = (s_hi - lo) // BLK + 1

    plsc.subcore_barrier()

    def cbody(j, carry):
        bj = lo + j * BLK

        def ibody(m, carry2):
            idx_vm[pl.ds(m * L, L)] = bj + m * L + iota16
            return carry2
        lax.fori_loop(0, BLK // L, ibody, 0)
        pltpu.sync_copy(acc_vm.at[pl.ds(bj, BLK)], shared.at[idx_vm], add=True)
        return carry
    lax.fori_loop(0, nblk, cbody, 0)

    plsc.subcore_barrier()

    @pl.when(s == 0)
    def _():
        pltpu.sync_copy(shared, out_h.at[c])


@functools.partial(jax.jit, static_argnames=("interpret",))
def _sc_call(att, w, sae, numbers, mol_idx, interpret=False):
    mesh = plsc.VectorSubcoreMesh(core_axis_name="c", subcore_axis_name="s",
                                  num_cores=NC, num_subcores=NS)
    f = pl.kernel(
        _sc_body,
        out_type=jax.ShapeDtypeStruct((NC, NSEG), jnp.float32),
        mesh=mesh,
        scratch_types=[
            pltpu.VMEM((EMB, ZPAD), jnp.float32),
            pltpu.VMEM((EMB,), jnp.float32),
            pltpu.VMEM((ZPAD,), jnp.float32),
            pltpu.VMEM((ZPAD * L,), jnp.float32),
            pltpu.VMEM((CHUNK,), jnp.int32),
            pltpu.VMEM((CHUNK + L,), jnp.int32),
            pltpu.VMEM((NSEG + L,), jnp.float32),
            pltpu.VMEM((BLK,), jnp.int32),
            pltpu.VMEM_SHARED((NSEG,), jnp.float32),
        ] + [pltpu.SemaphoreType.DMA] * (2 * PIECES),
        compiler_params=pltpu.CompilerParams(needs_layout_passes=False),
        interpret=interpret,
    )
    return f(att, w, sae, numbers, mol_idx)


def kernel(numbers, mol_idx, charge, atom_table, w, sae_tensor):
    del charge
    att = jnp.zeros((EMB, ZPAD), jnp.float32).at[:, :NZ].set(atom_table.T)
    sae = sae_tensor[:ZPAD]
    parts = _sc_call(att, w, sae, numbers, mol_idx)
    return parts[0] + parts[1]


# speedup vs baseline: 1.1248x; 1.0273x over previous
import functools

import jax
import jax.numpy as jnp
from jax import lax
from jax.experimental import pallas as pl
from jax.experimental.pallas import tpu as pltpu
from jax.experimental.pallas import tpu_sc as plsc

NSEG = 16384
N_ATOMS = 1048576
EMB = 64
NZ = 119
ZPAD = 128
NC, NS, L = 2, 16, 16
NW = NC * NS
CHUNK = N_ATOMS // NW
NVEC = CHUNK // L
BLK = 512


UNROLL = 16


def _sc_body(att_h, w_h, sae_h, num_h, mol_h, out_h,
             att_vm, w_vm, sae_vm, ctab_vm, nums_vm, mols_vm,
             acc_vm, idx_vm, shared, sem_n, sem_m):
    c = lax.axis_index("c")
    s = lax.axis_index("s")
    base = (s * NC + c) * CHUNK

    cp_n = pltpu.make_async_copy(num_h.at[pl.ds(base, CHUNK)], nums_vm, sem_n)
    cp_m = pltpu.make_async_copy(mol_h.at[pl.ds(base, CHUNK)],
                                 mols_vm.at[pl.ds(0, CHUNK)], sem_m)
    cp_n.start()
    cp_m.start()

    pltpu.sync_copy(att_h, att_vm)
    pltpu.sync_copy(w_h, w_vm)
    pltpu.sync_copy(sae_h, sae_vm)

    accs = [jnp.zeros((L,), jnp.float32) for _ in range(ZPAD // L)]
    for db in range(EMB // L):
        wv = w_vm[pl.ds(db * L, L)]
        for j in range(L):
            ws = wv[j]
            d = db * L + j
            for zb in range(ZPAD // L):
                accs[zb] = accs[zb] + att_vm[d, pl.ds(zb * L, L)] * ws
    for zb in range(ZPAD // L):
        v = accs[zb] + sae_vm[pl.ds(zb * L, L)]
        for j in range(L):
            ctab_vm[pl.ds((zb * L + j) * L, L)] = jnp.broadcast_to(v[j], (L,))

    zero16 = jnp.zeros((L,), jnp.float32)

    @plsc.parallel_loop(0, NSEG // L, unroll=UNROLL)
    def _(i):
        acc_vm[pl.ds(i * L, L)] = zero16

    @pl.when(s == 0)
    def _():
        pltpu.sync_copy(acc_vm.at[pl.ds(0, NSEG)], shared)

    cp_m.wait()
    mols_vm[pl.ds(CHUNK, L)] = jnp.full((L,), NSEG, jnp.int32)
    s_lo = jnp.min(mols_vm[pl.ds(0, L)])
    s_hi = jnp.max(mols_vm[pl.ds(CHUNK - L, L)])
    lo = (s_lo // BLK) * BLK
    nblk = (s_hi - lo) // BLK + 1

    cp_n.wait()

    iota16 = lax.iota(jnp.int32, L)

    @plsc.parallel_loop(0, NVEC, unroll=UNROLL,
                        carry=jnp.zeros((L,), jnp.float32))
    def _(i, run):
        o = i * L
        nums = nums_vm[pl.ds(o, L)]
        mols = mols_vm[pl.ds(o, L)]
        moln = mols_vm[pl.ds(o + 1, L)]
        vals = plsc.load_gather(ctab_vm, [nums * L + iota16])
        p = plsc.cumsum(vals)
        cum = p + run
        m = mols != moln
        plsc.addupdate_scatter(acc_vm, [mols], cum, mask=m)
        plsc.addupdate_scatter(acc_vm, [moln], -cum, mask=m)
        return run + jnp.broadcast_to(p[L - 1], (L,))

    plsc.subcore_barrier()

    def cbody(j, carry):
        bj = lo + j * BLK
        for m in range(BLK // L):
            idx_vm[pl.ds(m * L, L)] = bj + m * L + iota16
        pltpu.sync_copy(acc_vm.at[pl.ds(bj, BLK)], shared.at[idx_vm], add=True)
        return carry
    lax.fori_loop(0, nblk, cbody, 0)

    plsc.subcore_barrier()

    @pl.when(s == 0)
    def _():
        pltpu.sync_copy(shared, out_h.at[c])


@functools.partial(jax.jit, static_argnames=("interpret",))
def _sc_call(att, w, sae, numbers, mol_idx, interpret=False):
    mesh = plsc.VectorSubcoreMesh(core_axis_name="c", subcore_axis_name="s",
                                  num_cores=NC, num_subcores=NS)
    f = pl.kernel(
        _sc_body,
        out_type=jax.ShapeDtypeStruct((NC, NSEG), jnp.float32),
        mesh=mesh,
        scratch_types=[
            pltpu.VMEM((EMB, ZPAD), jnp.float32),
            pltpu.VMEM((EMB,), jnp.float32),
            pltpu.VMEM((ZPAD,), jnp.float32),
            pltpu.VMEM((ZPAD * L,), jnp.float32),
            pltpu.VMEM((CHUNK,), jnp.int32),
            pltpu.VMEM((CHUNK + L,), jnp.int32),
            pltpu.VMEM((NSEG + L,), jnp.float32),
            pltpu.VMEM((BLK,), jnp.int32),
            pltpu.VMEM_SHARED((NSEG,), jnp.float32),
            pltpu.SemaphoreType.DMA,
            pltpu.SemaphoreType.DMA,
        ],
        compiler_params=pltpu.CompilerParams(needs_layout_passes=False),
        interpret=interpret,
    )
    return f(att, w, sae, numbers, mol_idx)


def kernel(numbers, mol_idx, charge, atom_table, w, sae_tensor):
    del charge
    att = jnp.zeros((EMB, ZPAD), jnp.float32).at[:, :NZ].set(atom_table.T)
    sae = sae_tensor[:ZPAD]
    parts = _sc_call(att, w, sae, numbers, mol_idx)
    return parts[0] + parts[1]
